# Initial kernel scaffold; baseline (speedup 1.0000x reference)
import functools

import jax
import jax.numpy as jnp
from jax import lax
from jax.experimental import pallas as pl
from jax.experimental.pallas import tpu as pltpu
from jax.experimental.pallas import tpu_sc as plsc

_NUM_FIELDS = 26
_FIELD_SIZE = 100000
_BATCH = 16384
_NC = 2
_NS = 16
_NW = _NC * _NS
_BPW = _BATCH // _NW
_LANES = 16
_IDX_MINOR = 128
_ROWS = _NUM_FIELDS * _BPW // _IDX_MINOR
_RPF = _BPW // _IDX_MINOR
_CHUNKS = _IDX_MINOR // _LANES


def _sc_embed_sum(xw, table_flat):
    mesh = plsc.VectorSubcoreMesh(core_axis_name="c", subcore_axis_name="s")

    @functools.partial(
        pl.kernel,
        out_type=jax.ShapeDtypeStruct((_BATCH,), jnp.float32),
        mesh=mesh,
        scratch_types=[
            pltpu.VMEM((_ROWS, _IDX_MINOR), jnp.int32),
            pltpu.VMEM((_ROWS, _IDX_MINOR), jnp.float32),
            pltpu.VMEM((_BPW,), jnp.float32),
            pltpu.SemaphoreType.DMA,
        ],
    )
    def k(xw_hbm, table_hbm, out_hbm, idx_v, vals_v, out_v, sem):
        wid = lax.axis_index("s") * _NC + lax.axis_index("c")
        pltpu.sync_copy(xw_hbm.at[wid], idx_v)

        def add_off(r, carry):
            off = (r // _RPF) * _FIELD_SIZE
            for c in range(_CHUNKS):
                sl = pl.ds(c * _LANES, _LANES)
                idx_v[r, sl] = idx_v[r, sl] + off
            return carry

        lax.fori_loop(0, _ROWS, add_off, 0)

        pltpu.async_copy(table_hbm.at[idx_v], vals_v, sem).wait()

        for q in range(_RPF):
            for c in range(_CHUNKS):
                sl = pl.ds(c * _LANES, _LANES)

                def body(f, acc):
                    return acc + vals_v[f * _RPF + q, sl]

                acc = lax.fori_loop(
                    0, _NUM_FIELDS, body, jnp.zeros((_LANES,), jnp.float32)
                )
                out_v[pl.ds(q * _IDX_MINOR + c * _LANES, _LANES)] = acc

        pltpu.sync_copy(out_v, out_hbm.at[pl.ds(wid * _BPW, _BPW)])

    return k(xw, table_flat)


def kernel(x, table):
    x = x.astype(jnp.int32)
    xw = (
        x.reshape(_NW, _BPW, _NUM_FIELDS)
        .transpose(0, 2, 1)
        .reshape(_NW, _ROWS, _IDX_MINOR)
    )
    out = _sc_embed_sum(xw, table.reshape(-1))
    return out.reshape(_BATCH, 1)


# speedup vs baseline: 1.2366x; 1.2366x over previous
import functools

import jax
import jax.numpy as jnp
from jax import lax
from jax.experimental import pallas as pl
from jax.experimental.pallas import tpu as pltpu
from jax.experimental.pallas import tpu_sc as plsc

_NUM_FIELDS = 26
_FIELD_SIZE = 100000
_BATCH = 16384
_NC = 2
_NS = 16
_NW = _NC * _NS
_BPW = _BATCH // _NW
_LANES = 16
_IPW = _NUM_FIELDS * _BPW
_VPF = _BPW // _LANES


def _sc_embed_sum(xw, table_flat):
    mesh = plsc.VectorSubcoreMesh(core_axis_name="c", subcore_axis_name="s")

    @functools.partial(
        pl.kernel,
        out_type=jax.ShapeDtypeStruct((_BATCH,), jnp.float32),
        mesh=mesh,
        scratch_types=[
            pltpu.VMEM((_IPW,), jnp.int32),
            pltpu.VMEM((_IPW,), jnp.float32),
            pltpu.VMEM((_BPW,), jnp.float32),
            pltpu.SemaphoreType.DMA,
        ],
    )
    def k(xw_hbm, table_hbm, out_hbm, idx_v, vals_v, out_v, sem):
        wid = lax.axis_index("s") * _NC + lax.axis_index("c")
        pltpu.sync_copy(xw_hbm.at[wid], idx_v)

        def add_off(p16, carry):
            off = (p16 // _VPF) * _FIELD_SIZE
            sl = pl.ds(p16 * _LANES, _LANES)
            idx_v[sl] = idx_v[sl] + off
            return carry

        lax.fori_loop(0, _IPW // _LANES, add_off, 0)

        pltpu.async_copy(table_hbm.at[idx_v], vals_v, sem).wait()

        for v in range(_VPF):
            base = v * _LANES

            def body(f, acc):
                return acc + vals_v[pl.ds(f * _BPW + base, _LANES)]

            acc = lax.fori_loop(
                0, _NUM_FIELDS, body, jnp.zeros((_LANES,), jnp.float32)
            )
            out_v[pl.ds(base, _LANES)] = acc

        pltpu.sync_copy(out_v, out_hbm.at[pl.ds(wid * _BPW, _BPW)])

    return k(xw, table_flat)


def kernel(x, table):
    x = x.astype(jnp.int32)
    xw = (
        x.reshape(_NW, _BPW, _NUM_FIELDS)
        .transpose(0, 2, 1)
        .reshape(_NW, _IPW)
    )
    out = _sc_embed_sum(xw, table.reshape(-1))
    return out.reshape(_BATCH, 1)
